# baseline (speedup 1.0000x reference)
import functools

import jax
import jax.numpy as jnp
from jax import lax
from jax.experimental import pallas as pl
from jax.experimental.pallas import tpu as pltpu
from jax.experimental.pallas import tpu_sc as plsc


def _route_body(T, E, K, ti_hbm, tw_hbm, cmb_hbm, ti_v, tw_v, row_v):
    c = lax.axis_index("c")
    s = lax.axis_index("s")

    @pl.when((c == 0) & (s < E))
    def _():
        e = s
        pltpu.sync_copy(ti_hbm, ti_v)
        pltpu.sync_copy(tw_hbm, tw_v)
        for i in range(T // 16):
            vals = jnp.zeros((16,), jnp.float32)
            for k in range(K):
                ids = ti_v[pl.ds(k * T + i * 16, 16)]
                tws = tw_v[pl.ds(k * T + i * 16, 16)]
                vals = vals + jnp.where(ids == e, tws, jnp.zeros((16,), jnp.float32))
            row_v[pl.ds(i * 16, 16)] = vals
        pltpu.sync_copy(row_v, cmb_hbm.at[pl.ds(e * T, T)])


def _route_combine_weights(topk_weights, topk_ids, T, E, K):
    mesh = plsc.VectorSubcoreMesh(core_axis_name="c", subcore_axis_name="s")
    route = functools.partial(
        pl.kernel,
        mesh=mesh,
        out_type=jax.ShapeDtypeStruct((E * T,), jnp.float32),
        scratch_types=[
            pltpu.VMEM((K * T,), jnp.int32),
            pltpu.VMEM((K * T,), jnp.float32),
            pltpu.VMEM((T,), jnp.float32),
        ],
    )(functools.partial(_route_body, T, E, K))
    cmb = route(topk_ids.T.reshape(-1), topk_weights.T.reshape(-1))
    return cmb.reshape(E, T)


def _ffn_body(x_ref, cmb_ref, wg_ref, wu_ref,
              w2a_ref, w2b_ref, w2c_ref, w2d_ref, out_ref):
    e = pl.program_id(0)
    Dq = w2a_ref.shape[1]

    @pl.when(e == 0)
    def _():
        out_ref[...] = jnp.zeros_like(out_ref)

    x = x_ref[...]

    def dott(a, b):
        return jax.lax.dot_general(a, b, (((1,), (1,)), ((), ())),
                                   preferred_element_type=jnp.float32)

    g = dott(x, wg_ref[0, 0])
    u = dott(x, wu_ref[0, 0])
    act = (g * jax.nn.sigmoid(g)) * u

    cmb = cmb_ref[...]
    erow = jax.lax.broadcasted_iota(jnp.int32, cmb.shape, 0)
    wvec = jnp.sum(jnp.where(erow == e, cmb, 0.0), axis=0)[:, None]

    for q, w2q in enumerate((w2a_ref, w2b_ref, w2c_ref, w2d_ref)):
        yq = dott(act, w2q[0])
        out_ref[:, q * Dq:(q + 1) * Dq] += wvec * yq


@jax.jit
def kernel(hidden_states, topk_weights, topk_ids, w1, w2):
    T, D = hidden_states.shape
    E = w1.shape[0]
    F = w1.shape[1] // 2
    K = topk_ids.shape[1]

    cmb = _route_combine_weights(topk_weights, topk_ids, T, E, K)

    w1r = w1.reshape(E, 2, F, D)

    grid = (E,)
    w1spec = lambda q: pl.BlockSpec((1, 1, F, D), lambda e, q=q: (e, q, 0, 0))
    w2spec = lambda q: pl.BlockSpec((1, D // 4, F), lambda e, q=q: (e, q, 0))
    out = pl.pallas_call(
        _ffn_body,
        grid=grid,
        in_specs=[
            pl.BlockSpec((T, D), lambda e: (0, 0)),
            pl.BlockSpec((E, T), lambda e: (0, 0)),
            w1spec(0), w1spec(1),
            w2spec(0), w2spec(1), w2spec(2), w2spec(3),
        ],
        out_specs=pl.BlockSpec((T, D), lambda e: (0, 0)),
        out_shape=jax.ShapeDtypeStruct((T, D), jnp.float32),
    )(hidden_states, cmb, w1r, w1r, w2, w2, w2, w2)
    return out


# speedup vs baseline: 1.2894x; 1.2894x over previous
import functools

import jax
import jax.numpy as jnp
from jax.experimental import pallas as pl


def _ffn_body(x_ref, tw_ref, ti_ref, wg_ref, wu_ref,
              w2a_ref, w2b_ref, w2c_ref, w2d_ref, out_ref):
    e = pl.program_id(0)
    Dq = w2a_ref.shape[1]

    @pl.when(e == 0)
    def _():
        out_ref[...] = jnp.zeros_like(out_ref)

    x = x_ref[...]

    def dott(a, b):
        return jax.lax.dot_general(a, b, (((1,), (1,)), ((), ())),
                                   preferred_element_type=jnp.float32)

    g = dott(x, wg_ref[0, 0])
    u = dott(x, wu_ref[0, 0])
    act = (g * jax.nn.sigmoid(g)) * u

    ids = ti_ref[...]
    tw = tw_ref[...]
    wvec = jnp.sum(jnp.where(ids == e, tw, 0.0), axis=1)[:, None]

    for q, w2q in enumerate((w2a_ref, w2b_ref, w2c_ref, w2d_ref)):
        yq = dott(act, w2q[0])
        out_ref[:, q * Dq:(q + 1) * Dq] += wvec * yq


@jax.jit
def kernel(hidden_states, topk_weights, topk_ids, w1, w2):
    T, D = hidden_states.shape
    E = w1.shape[0]
    F = w1.shape[1] // 2

    w1r = w1.reshape(E, 2, F, D)

    grid = (E,)
    w1spec = lambda q: pl.BlockSpec((1, 1, F, D), lambda e, q=q: (e, q, 0, 0))
    w2spec = lambda q: pl.BlockSpec((1, D // 4, F), lambda e, q=q: (e, q, 0))
    out = pl.pallas_call(
        _ffn_body,
        grid=grid,
        in_specs=[
            pl.BlockSpec((T, D), lambda e: (0, 0)),
            pl.BlockSpec(topk_weights.shape, lambda e: (0, 0)),
            pl.BlockSpec(topk_ids.shape, lambda e: (0, 0)),
            w1spec(0), w1spec(1),
            w2spec(0), w2spec(1), w2spec(2), w2spec(3),
        ],
        out_specs=pl.BlockSpec((T, D), lambda e: (0, 0)),
        out_shape=jax.ShapeDtypeStruct((T, D), jnp.float32),
    )(hidden_states, topk_weights, topk_ids,
      w1r, w1r, w2, w2, w2, w2)
    return out
